# Initial kernel scaffold; baseline (speedup 1.0000x reference)
import functools

import jax
import jax.numpy as jnp
from jax import lax
from jax.experimental import pallas as pl
from jax.experimental.pallas import tpu as pltpu
from jax.experimental.pallas import tpu_sc as plsc

M = 1000000
B = 16384
ALPHA = 0.9

NC = 2
NS = 16
NW = NC * NS
L = 16

CH = 31256
CH_LAST = M - (NW - 1) * CH
assert CH % 8 == 0 and CH_LAST % 8 == 0 and CH_LAST <= CH
NB = B // L


def _ema_body(data_hbm, idx_hbm, par_hbm, upd_hbm, out_hbm,
              pch, och, uch, idxv, datav):
    wid = lax.axis_index("s") * NC + lax.axis_index("c")
    lo = wid * CH
    is_last = wid == NW - 1

    pltpu.sync_copy(idx_hbm, idxv)
    pltpu.sync_copy(data_hbm, datav)

    @pl.when(jnp.logical_not(is_last))
    def _():
        src = par_hbm.at[pl.ds(lo, CH)]
        pltpu.sync_copy(src, pch)
        pltpu.sync_copy(src, och)
        pltpu.sync_copy(upd_hbm.at[pl.ds(lo, CH)], uch)

    @pl.when(is_last)
    def _():
        src = par_hbm.at[pl.ds(lo, CH_LAST)]
        pltpu.sync_copy(src, pch.at[pl.ds(0, CH_LAST)])
        pltpu.sync_copy(src, och.at[pl.ds(0, CH_LAST)])
        pltpu.sync_copy(upd_hbm.at[pl.ds(lo, CH_LAST)], uch.at[pl.ds(0, CH_LAST)])

    hi = lo + jnp.where(is_last, CH_LAST, CH)

    def step(i, _):
        off = i * L
        idx = idxv[pl.ds(off, L)]
        d = datav[pl.ds(off, L)]
        m = jnp.logical_and(idx >= lo, idx < hi)
        loc = jnp.where(m, idx - lo, 0)
        p = plsc.load_gather(pch, [loc], mask=m)
        u = plsc.load_gather(uch, [loc], mask=m)
        nv = ALPHA * p + (1.0 - ALPHA * u) * d
        plsc.store_scatter(och, [loc], nv, mask=m)
        return _

    lax.fori_loop(0, NB, step, None)

    @pl.when(jnp.logical_not(is_last))
    def _():
        pltpu.sync_copy(och, out_hbm.at[pl.ds(lo, CH)])

    @pl.when(is_last)
    def _():
        pltpu.sync_copy(och.at[pl.ds(0, CH_LAST)], out_hbm.at[pl.ds(lo, CH_LAST)])


@jax.jit
def _ema_update(data, index, parameter, updated):
    mesh = plsc.VectorSubcoreMesh(core_axis_name="c", subcore_axis_name="s",
                                  num_cores=NC, num_subcores=NS)
    return pl.kernel(
        _ema_body,
        out_type=jax.ShapeDtypeStruct((M,), jnp.float32),
        mesh=mesh,
        scratch_types=[
            pltpu.VMEM((CH,), jnp.float32),
            pltpu.VMEM((CH,), jnp.float32),
            pltpu.VMEM((CH,), jnp.float32),
            pltpu.VMEM((B,), jnp.int32),
            pltpu.VMEM((B,), jnp.float32),
        ],
    )(data, index, parameter, updated)


def kernel(data, index, parameter, updated):
    return _ema_update(data, index, parameter, updated)


# speedup vs baseline: 1.9751x; 1.9751x over previous
import functools

import jax
import jax.numpy as jnp
from jax import lax
from jax.experimental import pallas as pl
from jax.experimental.pallas import tpu as pltpu
from jax.experimental.pallas import tpu_sc as plsc

M = 1000000
B = 16384
ALPHA = 0.9

NC = 2
NS = 16
NW = NC * NS
L = 16

CH = 31256
CH_LAST = M - (NW - 1) * CH
assert CH % 8 == 0 and CH_LAST % 8 == 0 and CH_LAST <= CH
NB = B // L


def _ema_body(data_hbm, idx_hbm, par_hbm, upd_hbm, out_hbm,
              pch, och, uch, idxv, datav):
    wid = lax.axis_index("s") * NC + lax.axis_index("c")
    lo = wid * CH
    is_last = wid == NW - 1

    pltpu.sync_copy(idx_hbm, idxv)
    pltpu.sync_copy(data_hbm, datav)

    @pl.when(jnp.logical_not(is_last))
    def _():
        src = par_hbm.at[pl.ds(lo, CH)]
        pltpu.sync_copy(src, pch)
        pltpu.sync_copy(src, och)
        pltpu.sync_copy(upd_hbm.at[pl.ds(lo, CH)], uch)

    @pl.when(is_last)
    def _():
        src = par_hbm.at[pl.ds(lo, CH_LAST)]
        pltpu.sync_copy(src, pch.at[pl.ds(0, CH_LAST)])
        pltpu.sync_copy(src, och.at[pl.ds(0, CH_LAST)])
        pltpu.sync_copy(upd_hbm.at[pl.ds(lo, CH_LAST)], uch.at[pl.ds(0, CH_LAST)])

    hi = lo + jnp.where(is_last, CH_LAST, CH)

    def step(i, _):
        off = i * L
        idx = idxv[pl.ds(off, L)]
        d = datav[pl.ds(off, L)]
        m = jnp.logical_and(idx >= lo, idx < hi)
        loc = jnp.where(m, idx - lo, 0)
        p = plsc.load_gather(pch, [loc], mask=m)
        u = plsc.load_gather(uch, [loc], mask=m)
        nv = ALPHA * p + (1.0 - ALPHA * u) * d
        plsc.store_scatter(och, [loc], nv, mask=m)
        return _

    lax.fori_loop(0, NB, step, None)

    @pl.when(jnp.logical_not(is_last))
    def _():
        pltpu.sync_copy(och, out_hbm.at[pl.ds(lo, CH)])

    @pl.when(is_last)
    def _():
        pltpu.sync_copy(och.at[pl.ds(0, CH_LAST)], out_hbm.at[pl.ds(lo, CH_LAST)])


@jax.jit
def _ema_update(data, index, parameter, updated):
    mesh = plsc.VectorSubcoreMesh(core_axis_name="c", subcore_axis_name="s",
                                  num_cores=NC, num_subcores=NS)
    return pl.kernel(
        _ema_body,
        out_type=jax.ShapeDtypeStruct((M,), jnp.float32),
        mesh=mesh,
        compiler_params=pltpu.CompilerParams(needs_layout_passes=False),
        scratch_types=[
            pltpu.VMEM((CH,), jnp.float32),
            pltpu.VMEM((CH,), jnp.float32),
            pltpu.VMEM((CH,), jnp.float32),
            pltpu.VMEM((B,), jnp.int32),
            pltpu.VMEM((B,), jnp.float32),
        ],
    )(data, index, parameter, updated)


def kernel(data, index, parameter, updated):
    return _ema_update(data, index, parameter, updated)


# speedup vs baseline: 2.0073x; 1.0163x over previous
import functools

import jax
import jax.numpy as jnp
from jax import lax
from jax.experimental import pallas as pl
from jax.experimental.pallas import tpu as pltpu
from jax.experimental.pallas import tpu_sc as plsc

M = 1000000
B = 16384
ALPHA = 0.9

NC = 2
NS = 16
NW = NC * NS
L = 16

CH = 31256
CH_LAST = M - (NW - 1) * CH
assert CH % 8 == 0 and CH_LAST % 8 == 0 and CH_LAST <= CH
NB = B // L


def _ema_body(data_hbm, idx_hbm, par_hbm, upd_hbm, out_hbm,
              pch, och, uch, idxv, datav):
    wid = lax.axis_index("s") * NC + lax.axis_index("c")
    lo = wid * CH
    is_last = wid == NW - 1

    pltpu.sync_copy(idx_hbm, idxv)
    pltpu.sync_copy(data_hbm, datav)

    @pl.when(jnp.logical_not(is_last))
    def _():
        src = par_hbm.at[pl.ds(lo, CH)]
        pltpu.sync_copy(src, pch)
        pltpu.sync_copy(src, och)
        pltpu.sync_copy(upd_hbm.at[pl.ds(lo, CH)], uch)

    @pl.when(is_last)
    def _():
        src = par_hbm.at[pl.ds(lo, CH_LAST)]
        pltpu.sync_copy(src, pch.at[pl.ds(0, CH_LAST)])
        pltpu.sync_copy(src, och.at[pl.ds(0, CH_LAST)])
        pltpu.sync_copy(upd_hbm.at[pl.ds(lo, CH_LAST)], uch.at[pl.ds(0, CH_LAST)])

    hi = lo + jnp.where(is_last, CH_LAST, CH)

    UNROLL = 8

    def step(i, _):
        base = i * (L * UNROLL)
        for k in range(UNROLL):
            off = base + k * L
            idx = idxv[pl.ds(off, L)]
            d = datav[pl.ds(off, L)]
            m = jnp.logical_and(idx >= lo, idx < hi)
            loc = jnp.where(m, idx - lo, 0)
            p = plsc.load_gather(pch, [loc], mask=m)
            u = plsc.load_gather(uch, [loc], mask=m)
            nv = ALPHA * p + (1.0 - ALPHA * u) * d
            plsc.store_scatter(och, [loc], nv, mask=m)
        return _

    lax.fori_loop(0, NB // UNROLL, step, None)

    @pl.when(jnp.logical_not(is_last))
    def _():
        pltpu.sync_copy(och, out_hbm.at[pl.ds(lo, CH)])

    @pl.when(is_last)
    def _():
        pltpu.sync_copy(och.at[pl.ds(0, CH_LAST)], out_hbm.at[pl.ds(lo, CH_LAST)])


@jax.jit
def _ema_update(data, index, parameter, updated):
    mesh = plsc.VectorSubcoreMesh(core_axis_name="c", subcore_axis_name="s",
                                  num_cores=NC, num_subcores=NS)
    return pl.kernel(
        _ema_body,
        out_type=jax.ShapeDtypeStruct((M,), jnp.float32),
        mesh=mesh,
        compiler_params=pltpu.CompilerParams(needs_layout_passes=False),
        scratch_types=[
            pltpu.VMEM((CH,), jnp.float32),
            pltpu.VMEM((CH,), jnp.float32),
            pltpu.VMEM((CH,), jnp.float32),
            pltpu.VMEM((B,), jnp.int32),
            pltpu.VMEM((B,), jnp.float32),
        ],
    )(data, index, parameter, updated)


def kernel(data, index, parameter, updated):
    return _ema_update(data, index, parameter, updated)


# speedup vs baseline: 2.1370x; 1.0646x over previous
import jax
import jax.numpy as jnp
from jax import lax
from jax.experimental import pallas as pl
from jax.experimental.pallas import tpu as pltpu
from jax.experimental.pallas import tpu_sc as plsc

M = 1000000
B = 16384
ALPHA = 0.9

NC = 2
NS = 16
NW = NC * NS
L = 16

CH = 31256
CH_LAST = M - (NW - 1) * CH
assert CH % 8 == 0 and CH_LAST % 8 == 0 and CH_LAST <= CH
NB = B // L
UNROLL = 8
NSTAG = 8
SEG = B // NSTAG


def _ema_body(data_hbm, idx_hbm, par_hbm, upd_hbm, out_hbm,
              pch, och, uch, idxv, datav, sem):
    wid = lax.axis_index("s") * NC + lax.axis_index("c")
    lo = wid * CH
    is_last = wid == NW - 1

    def batch_copies():
        cps = []
        for j in range(NSTAG):
            part = lax.rem(wid + j, NSTAG)
            off = part * SEG
            cps.append(pltpu.make_async_copy(
                idx_hbm.at[pl.ds(off, SEG)], idxv.at[pl.ds(off, SEG)], sem))
            cps.append(pltpu.make_async_copy(
                data_hbm.at[pl.ds(off, SEG)], datav.at[pl.ds(off, SEG)], sem))
        return cps

    def chunk_copies(n):
        src = par_hbm.at[pl.ds(lo, n)]
        return [
            pltpu.make_async_copy(src, pch.at[pl.ds(0, n)], sem),
            pltpu.make_async_copy(src, och.at[pl.ds(0, n)], sem),
            pltpu.make_async_copy(upd_hbm.at[pl.ds(lo, n)],
                                  uch.at[pl.ds(0, n)], sem),
        ]

    @pl.when(jnp.logical_not(is_last))
    def _():
        for c in chunk_copies(CH):
            c.start()

    @pl.when(is_last)
    def _():
        for c in chunk_copies(CH_LAST):
            c.start()

    for c in batch_copies():
        c.start()
    for c in batch_copies():
        c.wait()

    @pl.when(jnp.logical_not(is_last))
    def _():
        for c in chunk_copies(CH):
            c.wait()

    @pl.when(is_last)
    def _():
        for c in chunk_copies(CH_LAST):
            c.wait()

    size_u = (jnp.where(is_last, CH_LAST, CH)).astype(jnp.uint32)

    def step(i, _):
        base = i * (L * UNROLL)
        for k in range(UNROLL):
            off = base + k * L
            idx = idxv[pl.ds(off, L)]
            d = datav[pl.ds(off, L)]
            loc = idx - lo
            m = loc.astype(jnp.uint32) < size_u
            p = plsc.load_gather(pch, [loc], mask=m)
            u = plsc.load_gather(uch, [loc], mask=m)
            nv = ALPHA * p + (1.0 - ALPHA * u) * d
            plsc.store_scatter(och, [loc], nv, mask=m)
        return _

    lax.fori_loop(0, NB // UNROLL, step, None)

    @pl.when(jnp.logical_not(is_last))
    def _():
        pltpu.sync_copy(och, out_hbm.at[pl.ds(lo, CH)])

    @pl.when(is_last)
    def _():
        pltpu.sync_copy(och.at[pl.ds(0, CH_LAST)], out_hbm.at[pl.ds(lo, CH_LAST)])


@jax.jit
def _ema_update(data, index, parameter, updated):
    mesh = plsc.VectorSubcoreMesh(core_axis_name="c", subcore_axis_name="s",
                                  num_cores=NC, num_subcores=NS)
    return pl.kernel(
        _ema_body,
        out_type=jax.ShapeDtypeStruct((M,), jnp.float32),
        mesh=mesh,
        compiler_params=pltpu.CompilerParams(needs_layout_passes=False),
        scratch_types=[
            pltpu.VMEM((CH,), jnp.float32),
            pltpu.VMEM((CH,), jnp.float32),
            pltpu.VMEM((CH,), jnp.float32),
            pltpu.VMEM((B,), jnp.int32),
            pltpu.VMEM((B,), jnp.float32),
            pltpu.SemaphoreType.DMA,
        ],
    )(data, index, parameter, updated)


def kernel(data, index, parameter, updated):
    return _ema_update(data, index, parameter, updated)


# speedup vs baseline: 2.3429x; 1.0963x over previous
import jax
import jax.numpy as jnp
from jax import lax
from jax.experimental import pallas as pl
from jax.experimental.pallas import tpu as pltpu
from jax.experimental.pallas import tpu_sc as plsc

M = 1000000
B = 16384
ALPHA = 0.9

NC = 2
NS = 16
NW = NC * NS
L = 16

CH = 31256
CH_LAST = M - (NW - 1) * CH
assert CH % 8 == 0 and CH_LAST % 8 == 0 and CH_LAST <= CH
NB = B // L
NSTAG = 8
SEG = B // NSTAG
ROUNDS = 4
VPR = NB // ROUNDS
PCAP = VPR * L + L
UNROLL_A = 8


def _ema_body(data_hbm, idx_hbm, par_hbm, upd_hbm, out_hbm,
              pch, och, uch, idxv, datav, posbuf, sem):
    wid = lax.axis_index("s") * NC + lax.axis_index("c")
    lo = wid * CH
    is_last = wid == NW - 1

    def batch_copies():
        cps = []
        for j in range(NSTAG):
            part = lax.rem(wid + j, NSTAG)
            off = part * SEG
            cps.append(pltpu.make_async_copy(
                idx_hbm.at[pl.ds(off, SEG)], idxv.at[pl.ds(off, SEG)], sem))
            cps.append(pltpu.make_async_copy(
                data_hbm.at[pl.ds(off, SEG)], datav.at[pl.ds(off, SEG)], sem))
        return cps

    def chunk_copies(n):
        src = par_hbm.at[pl.ds(lo, n)]
        return [
            pltpu.make_async_copy(src, pch.at[pl.ds(0, n)], sem),
            pltpu.make_async_copy(src, och.at[pl.ds(0, n)], sem),
            pltpu.make_async_copy(upd_hbm.at[pl.ds(lo, n)],
                                  uch.at[pl.ds(0, n)], sem),
        ]

    @pl.when(jnp.logical_not(is_last))
    def _():
        for c in chunk_copies(CH):
            c.start()

    @pl.when(is_last)
    def _():
        for c in chunk_copies(CH_LAST):
            c.start()

    for c in batch_copies():
        c.start()
    for c in batch_copies():
        c.wait()

    @pl.when(jnp.logical_not(is_last))
    def _():
        for c in chunk_copies(CH):
            c.wait()

    @pl.when(is_last)
    def _():
        for c in chunk_copies(CH_LAST):
            c.wait()

    size_u = (jnp.where(is_last, CH_LAST, CH)).astype(jnp.uint32)
    lane = lax.iota(jnp.int32, L)

    def filter_step(r):
        def stepA(i, cursor):
            base = (r * VPR + i * UNROLL_A) * L
            for k in range(UNROLL_A):
                off = base + k * L
                idx = idxv[pl.ds(off, L)]
                m = (idx - lo).astype(jnp.uint32) < size_u
                plsc.store_compressed(posbuf.at[pl.ds(cursor, L)],
                                      lane + off, mask=m)
                cursor = cursor + plsc.all_reduce_population_count(m)[0]
            return cursor

        return lax.fori_loop(0, VPR // UNROLL_A, stepA, jnp.int32(0))

    def apply_step(v, n):
        boff = v * L
        mB = (lane + boff) < n
        pos = posbuf[pl.ds(boff, L)]
        posc = jnp.where(mB, pos, 0)
        idx = plsc.load_gather(idxv, [posc], mask=mB)
        d = plsc.load_gather(datav, [posc], mask=mB)
        loc = idx - lo
        p = plsc.load_gather(pch, [loc], mask=mB)
        u = plsc.load_gather(uch, [loc], mask=mB)
        nv = ALPHA * p + (1.0 - ALPHA * u) * d
        plsc.store_scatter(och, [loc], nv, mask=mB)
        return n

    def round_step(r, _):
        n = filter_step(r)
        lax.fori_loop(0, (n + (L - 1)) // L, apply_step, n)
        return _

    lax.fori_loop(0, ROUNDS, round_step, None)

    @pl.when(jnp.logical_not(is_last))
    def _():
        pltpu.sync_copy(och, out_hbm.at[pl.ds(lo, CH)])

    @pl.when(is_last)
    def _():
        pltpu.sync_copy(och.at[pl.ds(0, CH_LAST)], out_hbm.at[pl.ds(lo, CH_LAST)])


@jax.jit
def _ema_update(data, index, parameter, updated):
    mesh = plsc.VectorSubcoreMesh(core_axis_name="c", subcore_axis_name="s",
                                  num_cores=NC, num_subcores=NS)
    return pl.kernel(
        _ema_body,
        out_type=jax.ShapeDtypeStruct((M,), jnp.float32),
        mesh=mesh,
        compiler_params=pltpu.CompilerParams(needs_layout_passes=False),
        scratch_types=[
            pltpu.VMEM((CH,), jnp.float32),
            pltpu.VMEM((CH,), jnp.float32),
            pltpu.VMEM((CH,), jnp.float32),
            pltpu.VMEM((B,), jnp.int32),
            pltpu.VMEM((B,), jnp.float32),
            pltpu.VMEM((PCAP,), jnp.int32),
            pltpu.SemaphoreType.DMA,
        ],
    )(data, index, parameter, updated)


def kernel(data, index, parameter, updated):
    return _ema_update(data, index, parameter, updated)
